# baseline (speedup 1.0000x reference)
import functools

import numpy as np
import jax
import jax.numpy as jnp
from jax import lax
from jax.experimental import pallas as pl
from jax.experimental.pallas import tpu as pltpu
from jax.experimental.pallas import tpu_sc as plsc

_SMOOTH = 0.001
_KAPPA = float(np.sqrt(3.0) / 8.0)

_NW = 32
_RPW = 8
_NR = 10
_W = 288
_NT = 17


def _ad(a, b):
    return jnp.abs(a - b)


def _row_vecs(buf, r, o_l, o_r):
    return ([buf[z, r, pl.ds(o_l, 16)] for z in range(4)],
            [buf[z, r, pl.ds(o_r, 16)] for z in range(4)])


def _row_edges(l, r):
    x = [_ad(l[z], r[z]) for z in range(4)]
    zl = [_ad(l[p], l[p + 1]) for p in range(3)]
    zr = [_ad(r[p], r[p + 1]) for p in range(3)]
    return x, zl, zr


def _areas(top, bot):
    (tl, tr, tx, tzl, tzr) = top
    (bl, br, bx, bzl, bzr) = bot
    xy = [
        tx[z] + bx[z] + _ad(tl[z], bl[z]) + _ad(tr[z], br[z])
        for z in range(4)
    ]
    return [
        xy[p] + xy[p + 1] + (tzl[p] + tzr[p] + bzl[p] + bzr[p])
        for p in range(3)
    ]


def _sc_body(pred_hbm, lab_hbm, out_hbm, praw, lraw, sig, lab, accv):
    wid = lax.axis_index("s") * 2 + lax.axis_index("c")
    row0 = wid * _RPW

    @pl.when(wid == 0)
    def _():
        for z in range(4):
            pltpu.sync_copy(pred_hbm.at[z, pl.ds(0, _NR - 1)],
                            praw.at[z, pl.ds(1, _NR - 1)])
            pltpu.sync_copy(lab_hbm.at[z, pl.ds(0, _NR - 1)],
                            lraw.at[z, pl.ds(1, _NR - 1)])

    @pl.when(wid == _NW - 1)
    def _():
        for z in range(4):
            pltpu.sync_copy(pred_hbm.at[z, pl.ds(256 - _NR + 1, _NR - 1)],
                            praw.at[z, pl.ds(0, _NR - 1)])
            pltpu.sync_copy(lab_hbm.at[z, pl.ds(256 - _NR + 1, _NR - 1)],
                            lraw.at[z, pl.ds(0, _NR - 1)])

    @pl.when(jnp.logical_and(wid > 0, wid < _NW - 1))
    def _():
        for z in range(4):
            pltpu.sync_copy(pred_hbm.at[z, pl.ds(row0 - 1, _NR)], praw.at[z])
            pltpu.sync_copy(lab_hbm.at[z, pl.ds(row0 - 1, _NR)], lraw.at[z])

    zeros = jnp.zeros((16,), jnp.float32)

    def _stage(u, carry):
        z = u // _NR
        m = u % _NR
        orig = row0 + m - 1
        valid = jnp.logical_and(orig >= 0, orig <= 255)
        for ref in (sig, lab):
            ref[z, m, pl.ds(0, 16)] = zeros
            ref[z, m, pl.ds(272, 16)] = zeros
        for k in range(16):
            v = praw[z, m, pl.ds(16 * k, 16)]
            sig[z, m, pl.ds(16 * k + 16, 16)] = jnp.where(
                valid, 1.0 / (1.0 + jnp.exp(-v)), zeros)
            w = lraw[z, m, pl.ds(16 * k, 16)].astype(jnp.float32)
            lab[z, m, pl.ds(16 * k + 16, 16)] = jnp.where(valid, w, zeros)
        return carry

    lax.fori_loop(0, 4 * _NR, _stage, 0)

    s8 = jnp.where(wid == _NW - 1, jnp.float32(1.0), jnp.float32(0.0))

    def _load_row(r, o_l, o_r):
        sl, sr = _row_vecs(sig, r, o_l, o_r)
        ll, lr = _row_vecs(lab, r, o_l, o_r)
        sx, szl, szr = _row_edges(sl, sr)
        lx, lzl, lzr = _row_edges(ll, lr)
        return (sl + sr + sx + szl + szr, ll + lr + lx + lzl + lzr)

    def _unpack(flat):
        return (flat[0:4], flat[4:8], flat[8:12], flat[12:15], flat[15:18])

    acc_n = zeros
    acc_d = zeros
    for t in range(_NT):
        o_l = 16 * t + 15
        o_r = 16 * t + 16

        def _row_step(rr, carry, o_l=o_l, o_r=o_r):
            acc_n, acc_d, stop, ltop = carry
            sbot, lbot = _load_row(rr + 1, o_l, o_r)
            pa3 = _areas(_unpack(stop), _unpack(sbot))
            la3 = _areas(_unpack(ltop), _unpack(lbot))
            s = jnp.where(rr < _RPW, jnp.float32(1.0), s8)
            for p in range(3):
                tot = pa3[p] + la3[p]
                both = jnp.logical_and(pa3[p] > 0.0, la3[p] > 0.0)
                acc_n = acc_n + s * jnp.where(both, tot, 0.0)
                acc_d = acc_d + s * tot
            return acc_n, acc_d, sbot, lbot

        stop0, ltop0 = _load_row(0, o_l, o_r)
        acc_n, acc_d, _, _ = lax.fori_loop(
            0, 9, _row_step, (acc_n, acc_d, stop0, ltop0))
    accv[0] = acc_n * _KAPPA
    accv[1] = acc_d * _KAPPA
    pltpu.sync_copy(accv, out_hbm.at[wid])


_sc_dice = functools.partial(
    pl.kernel,
    out_type=jax.ShapeDtypeStruct((_NW, 2, 16), jnp.float32),
    mesh=plsc.VectorSubcoreMesh(core_axis_name="c", subcore_axis_name="s"),
    scratch_types=[
        pltpu.VMEM((4, _NR, 256), jnp.float32),
        pltpu.VMEM((4, _NR, 256), jnp.int32),
        pltpu.VMEM((4, _NR, _W), jnp.float32),
        pltpu.VMEM((4, _NR, _W), jnp.float32),
        pltpu.VMEM((2, 16), jnp.float32),
    ],
    compiler_params=pltpu.CompilerParams(use_tc_tiling_on_sc=False),
)(_sc_body)


def _combine_body(parts_ref, out_ref):
    parts = parts_ref[...]
    n = jnp.sum(parts[:, 0, :])
    d = jnp.sum(parts[:, 1, :])
    dice = 1.0 - (n + _SMOOTH) / (d + _SMOOTH)
    out_ref[...] = jnp.full((1, 1), dice, jnp.float32)


def kernel(pred, labels):
    B = pred.shape[0]
    dices = []
    for b in range(B):
        parts = _sc_dice(pred[b], labels[b])
        out = pl.pallas_call(
            _combine_body,
            out_shape=jax.ShapeDtypeStruct((1, 1), jnp.float32),
        )(parts)
        dices.append(out[0, 0])
    return jnp.mean(jnp.stack(dices))


# speedup vs baseline: 1.0395x; 1.0395x over previous
import functools

import numpy as np
import jax
import jax.numpy as jnp
from jax import lax
from jax.experimental import pallas as pl
from jax.experimental.pallas import tpu as pltpu
from jax.experimental.pallas import tpu_sc as plsc

_SMOOTH = 0.001
_KAPPA = float(np.sqrt(3.0) / 8.0)

_NW = 32
_RPW = 8
_NR = 10
_W = 288
_NT = 17


def _ad(a, b):
    return jnp.abs(a - b)


def _edge_sum(buf, p, rr, o_l, o_r):
    a00 = buf[p, rr, pl.ds(o_l, 16)]
    a01 = buf[p, rr, pl.ds(o_r, 16)]
    a10 = buf[p, rr + 1, pl.ds(o_l, 16)]
    a11 = buf[p, rr + 1, pl.ds(o_r, 16)]
    b00 = buf[p + 1, rr, pl.ds(o_l, 16)]
    b01 = buf[p + 1, rr, pl.ds(o_r, 16)]
    b10 = buf[p + 1, rr + 1, pl.ds(o_l, 16)]
    b11 = buf[p + 1, rr + 1, pl.ds(o_r, 16)]
    return (
        _ad(a00, a01) + _ad(a10, a11) + _ad(b00, b01) + _ad(b10, b11)
        + _ad(a00, a10) + _ad(a01, a11) + _ad(b00, b10) + _ad(b01, b11)
        + _ad(a00, b00) + _ad(a01, b01) + _ad(a10, b10) + _ad(a11, b11)
    )


def _sc_body(pred_hbm, lab_hbm, out_hbm, praw, lraw, sig, lab, accv):
    wid = lax.axis_index("s") * 2 + lax.axis_index("c")
    row0 = wid * _RPW

    @pl.when(wid == 0)
    def _():
        for z in range(4):
            pltpu.sync_copy(pred_hbm.at[z, pl.ds(0, _NR - 1)],
                            praw.at[z, pl.ds(1, _NR - 1)])
            pltpu.sync_copy(lab_hbm.at[z, pl.ds(0, _NR - 1)],
                            lraw.at[z, pl.ds(1, _NR - 1)])

    @pl.when(wid == _NW - 1)
    def _():
        for z in range(4):
            pltpu.sync_copy(pred_hbm.at[z, pl.ds(256 - _NR + 1, _NR - 1)],
                            praw.at[z, pl.ds(0, _NR - 1)])
            pltpu.sync_copy(lab_hbm.at[z, pl.ds(256 - _NR + 1, _NR - 1)],
                            lraw.at[z, pl.ds(0, _NR - 1)])

    @pl.when(jnp.logical_and(wid > 0, wid < _NW - 1))
    def _():
        for z in range(4):
            pltpu.sync_copy(pred_hbm.at[z, pl.ds(row0 - 1, _NR)], praw.at[z])
            pltpu.sync_copy(lab_hbm.at[z, pl.ds(row0 - 1, _NR)], lraw.at[z])

    zeros = jnp.zeros((16,), jnp.float32)

    def _stage(u, carry):
        z = u // _NR
        m = u % _NR
        orig = row0 + m - 1
        valid = jnp.logical_and(orig >= 0, orig <= 255)
        for ref in (sig, lab):
            ref[z, m, pl.ds(0, 16)] = zeros
            ref[z, m, pl.ds(272, 16)] = zeros
        for k in range(16):
            v = praw[z, m, pl.ds(16 * k, 16)]
            sig[z, m, pl.ds(16 * k + 16, 16)] = jnp.where(
                valid, 1.0 / (1.0 + jnp.exp(-v)), zeros)
            w = lraw[z, m, pl.ds(16 * k, 16)].astype(jnp.float32)
            lab[z, m, pl.ds(16 * k + 16, 16)] = jnp.where(valid, w, zeros)
        return carry

    lax.fori_loop(0, 4 * _NR, _stage, 0)

    s8 = jnp.where(wid == _NW - 1, jnp.float32(1.0), jnp.float32(0.0))

    def _q_step(q, carry):
        acc_n, acc_d = carry
        rr = q // 3
        p = q % 3
        s = jnp.where(rr < _RPW, jnp.float32(1.0), s8)
        for t in range(_NT):
            o_l = 16 * t + 15
            o_r = 16 * t + 16
            pa = _edge_sum(sig, p, rr, o_l, o_r)
            la = _edge_sum(lab, p, rr, o_l, o_r)
            tot = pa + la
            both = jnp.logical_and(pa > 0.0, la > 0.0)
            acc_n = acc_n + s * jnp.where(both, tot, 0.0)
            acc_d = acc_d + s * tot
        return acc_n, acc_d

    acc_n, acc_d = lax.fori_loop(0, 27, _q_step, (zeros, zeros))
    accv[0] = acc_n * _KAPPA
    accv[1] = acc_d * _KAPPA
    pltpu.sync_copy(accv, out_hbm.at[wid])


_sc_dice = functools.partial(
    pl.kernel,
    out_type=jax.ShapeDtypeStruct((_NW, 2, 16), jnp.float32),
    mesh=plsc.VectorSubcoreMesh(core_axis_name="c", subcore_axis_name="s"),
    scratch_types=[
        pltpu.VMEM((4, _NR, 256), jnp.float32),
        pltpu.VMEM((4, _NR, 256), jnp.int32),
        pltpu.VMEM((4, _NR, _W), jnp.float32),
        pltpu.VMEM((4, _NR, _W), jnp.float32),
        pltpu.VMEM((2, 16), jnp.float32),
    ],
    compiler_params=pltpu.CompilerParams(use_tc_tiling_on_sc=False),
)(_sc_body)


def _combine_body(parts_ref, out_ref):
    parts = parts_ref[...]
    n = jnp.sum(parts[:, 0, :])
    d = jnp.sum(parts[:, 1, :])
    dice = 1.0 - (n + _SMOOTH) / (d + _SMOOTH)
    out_ref[...] = jnp.full((1, 1), dice, jnp.float32)


def kernel(pred, labels):
    B = pred.shape[0]
    dices = []
    for b in range(B):
        parts = _sc_dice(pred[b], labels[b])
        out = pl.pallas_call(
            _combine_body,
            out_shape=jax.ShapeDtypeStruct((1, 1), jnp.float32),
        )(parts)
        dices.append(out[0, 0])
    return jnp.mean(jnp.stack(dices))


# speedup vs baseline: 1.5195x; 1.4617x over previous
import functools

import numpy as np
import jax
import jax.numpy as jnp
from jax import lax
from jax.experimental import pallas as pl
from jax.experimental.pallas import tpu as pltpu
from jax.experimental.pallas import tpu_sc as plsc

_SMOOTH = 0.001
_KAPPA = float(np.sqrt(3.0) / 8.0)

_NW = 32
_ROWS_PER_W = 8
_NR = 10
_W = 288
_NT = 17
_NEG = -1.0e30


def _edge_sum(buf, p, rr, off0):
    a00 = buf[p, rr, pl.ds(off0, 16)]
    a01 = buf[p, rr, pl.ds(off0 + 1, 16)]
    a10 = buf[p, rr + 1, pl.ds(off0, 16)]
    a11 = buf[p, rr + 1, pl.ds(off0 + 1, 16)]
    b00 = buf[p + 1, rr, pl.ds(off0, 16)]
    b01 = buf[p + 1, rr, pl.ds(off0 + 1, 16)]
    b10 = buf[p + 1, rr + 1, pl.ds(off0, 16)]
    b11 = buf[p + 1, rr + 1, pl.ds(off0 + 1, 16)]
    return (
        jnp.abs(a00 - a01) + jnp.abs(a10 - a11)
        + jnp.abs(b00 - b01) + jnp.abs(b10 - b11)
        + jnp.abs(a00 - a10) + jnp.abs(a01 - a11)
        + jnp.abs(b00 - b10) + jnp.abs(b01 - b11)
        + jnp.abs(a00 - b00) + jnp.abs(a01 - b01)
        + jnp.abs(a10 - b10) + jnp.abs(a11 - b11)
    )


def _sc_body(pred_hbm, lab_hbm, out_hbm, praw, sig, lbuf, accv, dma_sem):
    wid = lax.axis_index("s") * 2 + lax.axis_index("c")
    row0 = wid * _ROWS_PER_W

    copies = []
    for z in range(4):
        copies.append(pltpu.async_copy(
            pred_hbm.at[z, pl.ds(row0, _NR)], praw.at[z], dma_sem))
        copies.append(pltpu.async_copy(
            lab_hbm.at[z, pl.ds(row0, _NR)], lbuf.at[z], dma_sem))
    for c in copies:
        c.wait()

    def _sig_pass(u, carry):
        z = u // _NR
        r = u % _NR
        for k in range(_W // 16):
            v = praw[z, r, pl.ds(16 * k, 16)]
            sig[z, r, pl.ds(16 * k, 16)] = 1.0 / (1.0 + jnp.exp(-v))
        return carry

    lax.fori_loop(0, 4 * _NR, _sig_pass, 0)

    def _row_pair(q, carry):
        acc_n, acc_d = carry
        rr = q // 3
        p = q % 3
        valid = jnp.logical_or(rr < _ROWS_PER_W, wid == _NW - 1)
        s = jnp.where(valid, jnp.float32(1.0), jnp.float32(0.0))
        for t in range(_NT):
            pa = _edge_sum(sig, p, rr, 16 * t)
            la = _edge_sum(lbuf, p, rr, 16 * t)
            tot = pa + la
            both = jnp.logical_and(pa > 0.0, la > 0.0)
            acc_n = acc_n + s * jnp.where(both, tot, 0.0)
            acc_d = acc_d + s * tot
        return acc_n, acc_d

    zeros = jnp.zeros((16,), jnp.float32)
    acc_n, acc_d = lax.fori_loop(0, 27, _row_pair, (zeros, zeros))
    accv[0] = acc_n * _KAPPA
    accv[1] = acc_d * _KAPPA
    pltpu.sync_copy(accv, out_hbm.at[wid])


_sc_dice = functools.partial(
    pl.kernel,
    out_type=jax.ShapeDtypeStruct((_NW, 2, 16), jnp.float32),
    mesh=plsc.VectorSubcoreMesh(core_axis_name="c", subcore_axis_name="s"),
    scratch_types=[
        pltpu.VMEM((4, _NR, _W), jnp.float32),
        pltpu.VMEM((4, _NR, _W), jnp.float32),
        pltpu.VMEM((4, _NR, _W), jnp.float32),
        pltpu.VMEM((2, 16), jnp.float32),
        pltpu.SemaphoreType.DMA,
    ],
    compiler_params=pltpu.CompilerParams(use_tc_tiling_on_sc=False),
)(_sc_body)


def _combine_body(parts_ref, out_ref):
    parts = parts_ref[...]
    n = jnp.sum(parts[:, 0, :])
    d = jnp.sum(parts[:, 1, :])
    dice = 1.0 - (n + _SMOOTH) / (d + _SMOOTH)
    out_ref[...] = jnp.full((1, 1), dice, jnp.float32)


def kernel(pred, labels):
    B = pred.shape[0]
    dices = []
    for b in range(B):
        pred_p = jnp.full((4, 258, _W), _NEG, jnp.float32)
        pred_p = pred_p.at[:, 1:257, 1:257].set(pred[b])
        lab_p = jnp.zeros((4, 258, _W), jnp.float32)
        lab_p = lab_p.at[:, 1:257, 1:257].set(labels[b].astype(jnp.float32))
        parts = _sc_dice(pred_p, lab_p)
        out = pl.pallas_call(
            _combine_body,
            out_shape=jax.ShapeDtypeStruct((1, 1), jnp.float32),
        )(parts)
        dices.append(out[0, 0])
    return jnp.mean(jnp.stack(dices))


# speedup vs baseline: 1.5280x; 1.0056x over previous
import functools

import numpy as np
import jax
import jax.numpy as jnp
from jax import lax
from jax.experimental import pallas as pl
from jax.experimental.pallas import tpu as pltpu
from jax.experimental.pallas import tpu_sc as plsc

_SMOOTH = 0.001
_KAPPA = float(np.sqrt(3.0) / 8.0)

_NW = 32
_ROWS_PER_W = 8
_NR = 10
_W = 288
_NT = 17
_NEG = -1.0e30


def _edge_sum(buf, p, rr, off0):
    a00 = buf[p, rr, pl.ds(off0, 16)]
    a01 = buf[p, rr, pl.ds(off0 + 1, 16)]
    a10 = buf[p, rr + 1, pl.ds(off0, 16)]
    a11 = buf[p, rr + 1, pl.ds(off0 + 1, 16)]
    b00 = buf[p + 1, rr, pl.ds(off0, 16)]
    b01 = buf[p + 1, rr, pl.ds(off0 + 1, 16)]
    b10 = buf[p + 1, rr + 1, pl.ds(off0, 16)]
    b11 = buf[p + 1, rr + 1, pl.ds(off0 + 1, 16)]
    return (
        jnp.abs(a00 - a01) + jnp.abs(a10 - a11)
        + jnp.abs(b00 - b01) + jnp.abs(b10 - b11)
        + jnp.abs(a00 - a10) + jnp.abs(a01 - a11)
        + jnp.abs(b00 - b10) + jnp.abs(b01 - b11)
        + jnp.abs(a00 - b00) + jnp.abs(a01 - b01)
        + jnp.abs(a10 - b10) + jnp.abs(a11 - b11)
    )


def _sc_body(pred_hbm, lab_hbm, out_hbm, praw, sig, lbuf, accv,
             pred_sem, lab_sem):
    wid = lax.axis_index("s") * 2 + lax.axis_index("c")
    row0 = wid * _ROWS_PER_W

    pred_copies = [
        pltpu.async_copy(pred_hbm.at[z, pl.ds(row0, _NR)], praw.at[z],
                         pred_sem)
        for z in range(4)
    ]
    lab_copies = [
        pltpu.async_copy(lab_hbm.at[z, pl.ds(row0, _NR)], lbuf.at[z],
                         lab_sem)
        for z in range(4)
    ]
    for c in pred_copies:
        c.wait()

    def _sig_pass(u, carry):
        z = u // _NR
        r = u % _NR
        for k in range(_W // 16):
            v = praw[z, r, pl.ds(16 * k, 16)]
            sig[z, r, pl.ds(16 * k, 16)] = 1.0 / (1.0 + jnp.exp(-v))
        return carry

    lax.fori_loop(0, 4 * _NR, _sig_pass, 0)
    for c in lab_copies:
        c.wait()

    def _row_pair(q, carry):
        acc_n, acc_d = carry
        rr = q // 3
        p = q % 3
        valid = jnp.logical_or(rr < _ROWS_PER_W, wid == _NW - 1)
        s = jnp.where(valid, jnp.float32(1.0), jnp.float32(0.0))
        for t in range(_NT):
            pa = _edge_sum(sig, p, rr, 16 * t)
            la = _edge_sum(lbuf, p, rr, 16 * t)
            tot = pa + la
            both = jnp.logical_and(pa > 0.0, la > 0.0)
            acc_n = acc_n + s * jnp.where(both, tot, 0.0)
            acc_d = acc_d + s * tot
        return acc_n, acc_d

    zeros = jnp.zeros((16,), jnp.float32)
    acc_n, acc_d = lax.fori_loop(0, 27, _row_pair, (zeros, zeros))
    accv[0] = acc_n * _KAPPA
    accv[1] = acc_d * _KAPPA
    pltpu.sync_copy(accv, out_hbm.at[wid])


_sc_dice = functools.partial(
    pl.kernel,
    out_type=jax.ShapeDtypeStruct((_NW, 2, 16), jnp.float32),
    mesh=plsc.VectorSubcoreMesh(core_axis_name="c", subcore_axis_name="s"),
    scratch_types=[
        pltpu.VMEM((4, _NR, _W), jnp.float32),
        pltpu.VMEM((4, _NR, _W), jnp.float32),
        pltpu.VMEM((4, _NR, _W), jnp.float32),
        pltpu.VMEM((2, 16), jnp.float32),
        pltpu.SemaphoreType.DMA,
        pltpu.SemaphoreType.DMA,
    ],
    compiler_params=pltpu.CompilerParams(use_tc_tiling_on_sc=False),
)(_sc_body)


def _combine_body(parts_ref, out_ref):
    parts = parts_ref[...]
    n = jnp.sum(parts[:, 0, :])
    d = jnp.sum(parts[:, 1, :])
    dice = 1.0 - (n + _SMOOTH) / (d + _SMOOTH)
    out_ref[...] = jnp.full((1, 1), dice, jnp.float32)


def kernel(pred, labels):
    B = pred.shape[0]
    dices = []
    for b in range(B):
        pred_p = jnp.full((4, 258, _W), _NEG, jnp.float32)
        pred_p = pred_p.at[:, 1:257, 1:257].set(pred[b])
        lab_p = jnp.zeros((4, 258, _W), jnp.float32)
        lab_p = lab_p.at[:, 1:257, 1:257].set(labels[b].astype(jnp.float32))
        parts = _sc_dice(pred_p, lab_p)
        out = pl.pallas_call(
            _combine_body,
            out_shape=jax.ShapeDtypeStruct((1, 1), jnp.float32),
        )(parts)
        dices.append(out[0, 0])
    return jnp.mean(jnp.stack(dices))
